# baseline (speedup 1.0000x reference)
import jax
import jax.numpy as jnp
from jax.experimental import pallas as pl

INPUT_DIM = 512
N_THRESH = 8
NUM_FACTS = INPUT_DIM * N_THRESH
BATCH = 16384
BLOCK_B = 512


def _body(x_ref, g_ref, th_ref, lk_ref, out_ref):
    xb = x_ref[...].astype(jnp.bfloat16)
    xg = jax.lax.dot_general(
        xb, g_ref[...], (((1,), (0,)), ((), ())),
        preferred_element_type=jnp.float32,
    )
    a = 0.5 * jnp.clip(jnp.exp(lk_ref[...]), 0.5, 50.0)
    c = a * th_ref[...]
    out_ref[...] = 0.5 + 0.5 * jnp.tanh(xg - c)


def kernel(x, th, log_kappa, feat_idx):
    g = (feat_idx[None, :] == jnp.arange(INPUT_DIM, dtype=feat_idx.dtype)[:, None])
    a = 0.5 * jnp.clip(jnp.exp(log_kappa), 0.5, 50.0)
    g = (g * a[None, :]).astype(jnp.bfloat16)
    th2 = th.reshape(1, NUM_FACTS)
    lk2 = log_kappa.reshape(1, NUM_FACTS)
    grid = (BATCH // BLOCK_B,)
    return pl.pallas_call(
        _body,
        grid=grid,
        in_specs=[
            pl.BlockSpec((BLOCK_B, INPUT_DIM), lambda i: (i, 0)),
            pl.BlockSpec((INPUT_DIM, NUM_FACTS), lambda i: (0, 0)),
            pl.BlockSpec((1, NUM_FACTS), lambda i: (0, 0)),
            pl.BlockSpec((1, NUM_FACTS), lambda i: (0, 0)),
        ],
        out_specs=pl.BlockSpec((BLOCK_B, NUM_FACTS), lambda i: (i, 0)),
        out_shape=jax.ShapeDtypeStruct((BATCH, NUM_FACTS), jnp.float32),
    )(x, g, th2, lk2)


# speedup vs baseline: 1.0462x; 1.0462x over previous
import jax
import jax.numpy as jnp
from jax.experimental import pallas as pl

INPUT_DIM = 512
N_THRESH = 8
NUM_FACTS = INPUT_DIM * N_THRESH
BATCH = 16384
BLOCK_B = 1024


def _body(x_ref, g_ref, th_ref, lk_ref, out_ref):
    xb = x_ref[...].astype(jnp.bfloat16)
    xg = jax.lax.dot_general(
        xb, g_ref[...], (((1,), (0,)), ((), ())),
        preferred_element_type=jnp.float32,
    )
    a = 0.5 * jnp.clip(jnp.exp(lk_ref[...]), 0.5, 50.0)
    c = a * th_ref[...]
    out_ref[...] = 0.5 + 0.5 * jnp.tanh(xg - c)


def kernel(x, th, log_kappa, feat_idx):
    g = (feat_idx[None, :] == jnp.arange(INPUT_DIM, dtype=feat_idx.dtype)[:, None])
    a = 0.5 * jnp.clip(jnp.exp(log_kappa), 0.5, 50.0)
    g = (g * a[None, :]).astype(jnp.bfloat16)
    th2 = th.reshape(1, NUM_FACTS)
    lk2 = log_kappa.reshape(1, NUM_FACTS)
    grid = (BATCH // BLOCK_B,)
    return pl.pallas_call(
        _body,
        grid=grid,
        in_specs=[
            pl.BlockSpec((BLOCK_B, INPUT_DIM), lambda i: (i, 0)),
            pl.BlockSpec((INPUT_DIM, NUM_FACTS), lambda i: (0, 0)),
            pl.BlockSpec((1, NUM_FACTS), lambda i: (0, 0)),
            pl.BlockSpec((1, NUM_FACTS), lambda i: (0, 0)),
        ],
        out_specs=pl.BlockSpec((BLOCK_B, NUM_FACTS), lambda i: (i, 0)),
        out_shape=jax.ShapeDtypeStruct((BATCH, NUM_FACTS), jnp.float32),
    )(x, g, th2, lk2)
